# Initial kernel scaffold; baseline (speedup 1.0000x reference)
import jax
import jax.numpy as jnp
from jax.experimental import pallas as pl


def kernel(features_1, features_2, edges_1, edges_2, W1, b1, W2, b2, W3, b3, att_W, ntn_W, ntn_Wb, ntn_bias, fc_W, fc_b, score_W, score_b):
    raise NotImplementedError("write your pallas kernel here")


# speedup vs baseline: 10.4294x; 10.4294x over previous
import functools

import jax
import jax.numpy as jnp
from jax import lax
from jax.experimental import pallas as pl
from jax.experimental.pallas import tpu as pltpu
from jax.experimental.pallas import tpu_sc as plsc

N = 10000
E = 320000
NN = 2 * N
NNP = 20480
EE = 2 * E
NC, NS = 2, 16
NW = NC * NS
EPT = EE // NW
K = 80
NCHUNK = EPT // K
RPT = NNP // NS
ZR = 128

_mesh = plsc.VectorSubcoreMesh(core_axis_name="c", subcore_axis_name="s")


def _deg_body(dst_hbm, parts_hbm, dbuf, ones_b, zbuf, acc_sh):
    c = lax.axis_index("c")
    s = lax.axis_index("s")
    wid = s * NC + c

    def _init(r, _):
        zbuf[r, :] = jnp.zeros((16,), jnp.float32)
        return 0

    lax.fori_loop(0, ZR, _init, 0)

    def _ones(r, _):
        ones_b[r, :] = jnp.ones((16,), jnp.float32)
        return 0

    lax.fori_loop(0, K, _ones, 0)

    for q in range(RPT // ZR):
        pltpu.sync_copy(zbuf, acc_sh.at[pl.ds(s * RPT + q * ZR, ZR)])
    plsc.subcore_barrier()

    def _chunk(ch, _):
        pltpu.sync_copy(dst_hbm.at[pl.ds(wid * EPT + ch * K, K)], dbuf)
        pltpu.sync_copy(ones_b, acc_sh.at[dbuf], add=True)
        return 0

    lax.fori_loop(0, NCHUNK, _chunk, 0)
    plsc.subcore_barrier()
    pltpu.sync_copy(acc_sh.at[pl.ds(s * RPT, RPT)],
                    parts_hbm.at[c, pl.ds(s * RPT, RPT)])


_sc_params = pltpu.CompilerParams(use_tc_tiling_on_sc=False)

_deg_pass = pl.kernel(
    _deg_body,
    out_type=jax.ShapeDtypeStruct((NC, NNP, 16), jnp.float32),
    mesh=_mesh,
    compiler_params=_sc_params,
    scratch_types=[
        pltpu.VMEM((K,), jnp.int32),
        pltpu.VMEM((K, 16), jnp.float32),
        pltpu.VMEM((ZR, 16), jnp.float32),
        pltpu.VMEM_SHARED((NNP, 16), jnp.float32),
    ],
)


def _layer_body(C, hs_hbm, src_hbm, dst_hbm, parts_hbm,
                sbuf, dbuf, rows, zbuf, acc_sh):
    c = lax.axis_index("c")
    s = lax.axis_index("s")
    wid = s * NC + c

    def _init(r, _):
        for j in range(C // 16):
            zbuf[r, pl.ds(j * 16, 16)] = jnp.zeros((16,), jnp.float32)
        return 0

    lax.fori_loop(0, ZR, _init, 0)
    for q in range(RPT // ZR):
        pltpu.sync_copy(zbuf, acc_sh.at[pl.ds(s * RPT + q * ZR, ZR)])
    plsc.subcore_barrier()

    def _chunk(ch, _):
        pltpu.sync_copy(src_hbm.at[pl.ds(wid * EPT + ch * K, K)], sbuf)
        pltpu.sync_copy(dst_hbm.at[pl.ds(wid * EPT + ch * K, K)], dbuf)
        pltpu.sync_copy(hs_hbm.at[sbuf], rows)
        pltpu.sync_copy(rows, acc_sh.at[dbuf], add=True)
        return 0

    lax.fori_loop(0, NCHUNK, _chunk, 0)
    plsc.subcore_barrier()
    pltpu.sync_copy(acc_sh.at[pl.ds(s * RPT, RPT)],
                    parts_hbm.at[c, pl.ds(s * RPT, RPT)])


def _make_layer_pass(C):
    return pl.kernel(
        functools.partial(_layer_body, C),
        out_type=jax.ShapeDtypeStruct((NC, NNP, C), jnp.float32),
        mesh=_mesh,
        compiler_params=_sc_params,
        scratch_types=[
            pltpu.VMEM((K,), jnp.int32),
            pltpu.VMEM((K,), jnp.int32),
            pltpu.VMEM((K, C), jnp.float32),
            pltpu.VMEM((ZR, C), jnp.float32),
            pltpu.VMEM_SHARED((NNP, C), jnp.float32),
        ],
    )


_layer_pass = {C: _make_layer_pass(C) for C in (64, 32, 16)}


def _tc0_body(x_ref, w_ref, dp_ref, hs_ref, dinv_ref):
    deg = dp_ref[0, :, 0:1] + dp_ref[1, :, 0:1] + 1.0
    dinv = lax.rsqrt(deg)
    z = jnp.dot(x_ref[...], w_ref[...], preferred_element_type=jnp.float32)
    hs_ref[...] = z * dinv
    dinv_ref[...] = dinv


def _tc_mid_body(parts_ref, hs_ref, dinv_ref, b_ref, w_ref, out_ref):
    dinv = dinv_ref[...]
    acc = parts_ref[0] + parts_ref[1] + hs_ref[...]
    xh = jnp.maximum(acc * dinv + b_ref[...][None, :], 0.0)
    out_ref[...] = jnp.dot(
        xh, w_ref[...], preferred_element_type=jnp.float32) * dinv


def _tc3_body(parts_ref, hs_ref, dinv_ref, b_ref, attw_ref, ntnw_ref,
              wb_ref, nbias_ref, fcw_ref, fcb_ref, sw_ref, sb_ref, out_ref):
    emb = ((parts_ref[0] + parts_ref[1] + hs_ref[...]) * dinv_ref[...]
           + b_ref[...][None, :])

    def pool(eg):
        m = jnp.mean(eg, axis=0)
        ctx = jnp.sum(m[:, None] * attw_ref[...], axis=0)
        tg = jnp.tanh(ctx)
        logits = jnp.dot(eg, tg[:, None],
                         preferred_element_type=jnp.float32)
        sig = jax.nn.sigmoid(logits)
        return jnp.sum(eg * sig, axis=0)

    e1 = pool(emb[:N])
    e2 = pool(emb[N:NN])
    pmat = e1[:, None] * e2[None, :]
    scoring = jnp.sum(jnp.sum(pmat[:, :, None] * ntnw_ref[...], axis=0),
                      axis=0)
    cat = jnp.concatenate([e1, e2])
    blk = jnp.sum(wb_ref[...] * cat[None, :], axis=1)
    scores = jnp.maximum(scoring + blk + nbias_ref[:, 0], 0.0)
    sv = jnp.tanh(jnp.sum(scores[:, None] * fcw_ref[...], axis=0)
                  + fcb_ref[...])
    val = jnp.sum(sv * sw_ref[:, 0]) + sb_ref[...]
    out_ref[...] = jax.nn.sigmoid(val)


def _tc0(x, w, dp):
    return pl.pallas_call(
        _tc0_body,
        out_shape=[jax.ShapeDtypeStruct((NNP, w.shape[1]), jnp.float32),
                   jax.ShapeDtypeStruct((NNP, 1), jnp.float32)],
    )(x, w, dp)


def _tc_mid(parts, hs, dinv, b, w):
    return pl.pallas_call(
        _tc_mid_body,
        out_shape=jax.ShapeDtypeStruct((NNP, w.shape[1]), jnp.float32),
    )(parts, hs, dinv, b, w)


def _tc3(parts, hs, dinv, b, attw, ntnw, wb, nbias, fcw, fcb, sw, sb):
    return pl.pallas_call(
        _tc3_body,
        out_shape=jax.ShapeDtypeStruct((1,), jnp.float32),
    )(parts, hs, dinv, b, attw, ntnw, wb, nbias, fcw, fcb, sw, sb)


def kernel(features_1, features_2, edges_1, edges_2, W1, b1, W2, b2, W3, b3,
           att_W, ntn_W, ntn_Wb, ntn_bias, fc_W, fc_b, score_W, score_b):
    x = jnp.concatenate([features_1, features_2], axis=0)
    x = jnp.pad(x, ((0, NNP - NN), (0, 0)))
    src = jnp.concatenate([edges_1[0], edges_2[0] + N])
    dst = jnp.concatenate([edges_1[1], edges_2[1] + N])

    degparts = _deg_pass(dst)
    hs1, dinv = _tc0(x, W1, degparts)
    parts1 = _layer_pass[64](hs1, src, dst)
    hs2 = _tc_mid(parts1, hs1, dinv, b1, W2)
    parts2 = _layer_pass[32](hs2, src, dst)
    hs3 = _tc_mid(parts2, hs2, dinv, b2, W3)
    parts3 = _layer_pass[16](hs3, src, dst)
    return _tc3(parts3, hs3, dinv, b3, att_W, ntn_W, ntn_Wb, ntn_bias,
                fc_W, fc_b, score_W, score_b)
